# Initial kernel scaffold; baseline (speedup 1.0000x reference)
import jax
import jax.numpy as jnp
from jax import lax
from jax.experimental import pallas as pl
from jax.experimental.pallas import tpu as pltpu
from jax.experimental.pallas import tpu_sc as plsc

_NC = 2
_NS = 16
_NW = _NC * _NS
_P = 16

_G = 80
_CH = 2000


def _mlp_body(x_ref, w1_ref, b1_ref, w2_ref, b2_ref, o_ref):
    h = jnp.dot(x_ref[...], w1_ref[...], preferred_element_type=jnp.float32)
    h = jnp.maximum(h + b1_ref[...], 0.0)
    o = jnp.dot(h, w2_ref[...], preferred_element_type=jnp.float32)
    o_ref[...] = o + b2_ref[...]


def _combine_body(a_ref, b_ref, o_ref):
    o_ref[...] = a_ref[...] + b_ref[...]


def kernel(nodes, edges, senders, receivers, bn_scale, bn_bias, bn_mean,
           bn_var, W1, b1, W2, b2):
    del senders
    f32 = jnp.float32
    num_nodes = nodes.shape[0]
    E, DE = edges.shape
    DH = W1.shape[1]

    s = bn_scale * lax.rsqrt(bn_var + 1e-5)
    t = bn_bias - bn_mean * s
    W1f = s[:, None] * W1
    b1f = b1 + t @ W1

    W1p = jnp.zeros((_P, _P), f32).at[:DE, :DH].set(W1f)
    W2p = jnp.zeros((_P, _P), f32).at[:DH, :DH].set(W2)
    b1p = jnp.zeros((_P,), f32).at[:DH].set(b1f)
    b2p = jnp.zeros((_P,), f32).at[:DH].set(b2)
    R = 128 // _P
    eye = jnp.eye(R, dtype=f32)
    W1big = jnp.kron(eye, W1p)
    W2big = jnp.kron(eye, W2p)
    b1big = jnp.tile(b1p, R)[None, :]
    b2big = jnp.tile(b2p, R)[None, :]

    ROWS = E * DE // 128
    BLK = 2000
    assert ROWS % BLK == 0
    x2 = edges.reshape(ROWS, 128)
    new_rows = pl.pallas_call(
        _mlp_body,
        grid=(ROWS // BLK,),
        in_specs=[
            pl.BlockSpec((BLK, 128), lambda i: (i, 0)),
            pl.BlockSpec((128, 128), lambda i: (0, 0)),
            pl.BlockSpec((1, 128), lambda i: (0, 0)),
            pl.BlockSpec((128, 128), lambda i: (0, 0)),
            pl.BlockSpec((1, 128), lambda i: (0, 0)),
        ],
        out_specs=pl.BlockSpec((BLK, 128), lambda i: (i, 0)),
        out_shape=jax.ShapeDtypeStruct((ROWS, 128), f32),
    )(x2, W1big, b1big, W2big, b2big)
    new_flat = new_rows.reshape(E, _P)

    EPW = E // _NW
    GPT = EPW // _G
    NCHUNK = EPW // _CH
    GPC = _CH // _G
    assert EPW % _G == 0 and EPW % _CH == 0 and _CH % _G == 0
    NP = ((num_nodes + _NS * 16 - 1) // (_NS * 16)) * (_NS * 16)
    ZR = NP // _NS

    idx2 = receivers.reshape(E // _G, _G)

    def sc_body(rows_hbm, idx_hbm, out_hbm, acc, rows_v, idx_v, zer_v):
        c = lax.axis_index("c")
        sub = lax.axis_index("s")
        wid = c * _NS + sub
        row0 = sub * ZR

        for i in range(16):
            zer_v[i, :] = jnp.zeros((_P,), f32)

        def zloop(j, carry):
            pltpu.sync_copy(zer_v, acc.at[pl.ds(row0 + j * 16, 16)])
            return carry

        lax.fori_loop(0, ZR // 16, zloop, 0)
        plsc.subcore_barrier()

        pltpu.sync_copy(idx_hbm.at[pl.ds(wid * GPT, GPT)], idx_v)

        ebase = wid * EPW
        for ck in range(NCHUNK):
            pltpu.sync_copy(rows_hbm.at[pl.ds(ebase + ck * _CH, _CH)], rows_v)

            def gloop(j, carry, ck=ck):
                pltpu.sync_copy(rows_v.at[pl.ds(j * _G, _G)],
                                acc.at[idx_v.at[ck * GPC + j]], add=True)
                return carry

            lax.fori_loop(0, GPC, gloop, 0)
        plsc.subcore_barrier()

        pltpu.sync_copy(acc.at[pl.ds(row0, ZR)],
                        out_hbm.at[pl.ds(c * NP + row0, ZR)])

    mesh = plsc.VectorSubcoreMesh(core_axis_name="c", subcore_axis_name="s")
    partial = pl.kernel(
        sc_body,
        out_type=jax.ShapeDtypeStruct((_NC * NP, _P), f32),
        mesh=mesh,
        scratch_types=[
            pltpu.VMEM_SHARED((NP, _P), f32),
            pltpu.VMEM((_CH, _P), f32),
            pltpu.VMEM((GPT, _G), jnp.int32),
            pltpu.VMEM((16, _P), f32),
        ],
    )(new_flat, idx2)

    FR = NP * _P // 128
    pa = partial[:NP].reshape(FR, 128)
    pb = partial[NP:].reshape(FR, 128)
    flat = pl.pallas_call(
        _combine_body,
        out_shape=jax.ShapeDtypeStruct((FR, 128), f32),
    )(pa, pb)
    return flat.reshape(NP, _P)[:num_nodes, :DH]


# speedup vs baseline: 2.4942x; 2.4942x over previous
import jax
import jax.numpy as jnp
from jax import lax
from jax.experimental import pallas as pl
from jax.experimental.pallas import tpu as pltpu
from jax.experimental.pallas import tpu_sc as plsc

_NC = 2
_NS = 16
_NW = _NC * _NS
_P = 16

_G = 128
_EPW = 10240
_CHE = 2560


def _mlp_body(x_ref, w1_ref, b1_ref, w2_ref, b2_ref, o_ref):
    h = jnp.dot(x_ref[...], w1_ref[...], preferred_element_type=jnp.float32)
    h = jnp.maximum(h + b1_ref[...], 0.0)
    o = jnp.dot(h, w2_ref[...], preferred_element_type=jnp.float32)
    o_ref[...] = o + b2_ref[...]


def _combine_body(a_ref, b_ref, o_ref):
    o_ref[...] = a_ref[...] + b_ref[...]


def kernel(nodes, edges, senders, receivers, bn_scale, bn_bias, bn_mean,
           bn_var, W1, b1, W2, b2):
    del senders
    f32 = jnp.float32
    num_nodes = nodes.shape[0]
    E, DE = edges.shape
    DH = W1.shape[1]

    s = bn_scale * lax.rsqrt(bn_var + 1e-5)
    t = bn_bias - bn_mean * s
    W1f = s[:, None] * W1
    b1f = b1 + t @ W1

    W1p = jnp.zeros((DE, _P), f32).at[:, :DH].set(W1f)
    W2p = jnp.zeros((_P, _P), f32).at[:DH, :DH].set(W2)
    b1p = jnp.zeros((1, _P), f32).at[0, :DH].set(b1f)
    b2p = jnp.zeros((1, _P), f32).at[0, :DH].set(b2)

    EP = _NW * _EPW
    assert E <= EP
    BLKE = 3200
    assert E % BLKE == 0
    new_rows = pl.pallas_call(
        _mlp_body,
        grid=(E // BLKE,),
        in_specs=[
            pl.BlockSpec((BLKE, DE), lambda i: (i, 0)),
            pl.BlockSpec((DE, _P), lambda i: (0, 0)),
            pl.BlockSpec((1, _P), lambda i: (0, 0)),
            pl.BlockSpec((_P, _P), lambda i: (0, 0)),
            pl.BlockSpec((1, _P), lambda i: (0, 0)),
        ],
        out_specs=pl.BlockSpec((BLKE, _P), lambda i: (i, 0)),
        out_shape=jax.ShapeDtypeStruct((EP, _P), f32),
    )(edges, W1p, b1p, W2p, b2p)

    NP = ((num_nodes + _NS * _P) // (_NS * _P)) * (_NS * _P)
    ZR = NP // _NS
    GPT = _EPW // _G
    NCHUNK = _EPW // _CHE
    GPC = _CHE // _G

    idx2 = jnp.concatenate(
        [receivers, jnp.full((EP - E,), num_nodes, jnp.int32)]
    ).reshape(EP // _G, _G)

    def sc_body(rows_hbm, idx_hbm, out_hbm, acc, rows_v, idx_v, zer_v,
                lsem, ssem):
        c = lax.axis_index("c")
        sub = lax.axis_index("s")
        wid = c * _NS + sub

        for i in range(64):
            zer_v[i, :] = jnp.zeros((_P,), f32)
        row0 = sub * ZR

        def zloop(j, carry):
            pltpu.sync_copy(zer_v, acc.at[pl.ds(row0 + j * 64, 64)])
            return carry

        lax.fori_loop(0, ZR // 64, zloop, 0)

        pltpu.sync_copy(idx_hbm.at[pl.ds(wid * GPT, GPT)], idx_v)
        plsc.subcore_barrier()

        ebase = wid * _EPW
        for ck in range(NCHUNK):
            pltpu.sync_copy(rows_hbm.at[pl.ds(ebase + ck * _CHE, _CHE)],
                            rows_v.at[0])

            def gloop(j, carry, ck=ck):
                pltpu.sync_copy(rows_v.at[0].at[pl.ds(j * _G, _G)],
                                acc.at[idx_v.at[ck * GPC + j]], add=True)
                return carry

            lax.fori_loop(0, GPC, gloop, 0)
        plsc.subcore_barrier()

        pltpu.sync_copy(acc.at[pl.ds(row0, ZR)],
                        out_hbm.at[pl.ds(c * NP + row0, ZR)])

    mesh = plsc.VectorSubcoreMesh(core_axis_name="c", subcore_axis_name="s")
    partial = pl.kernel(
        sc_body,
        out_type=jax.ShapeDtypeStruct((_NC * NP, _P), f32),
        mesh=mesh,
        compiler_params=pltpu.CompilerParams(use_tc_tiling_on_sc=False),
        scratch_types=[
            pltpu.VMEM_SHARED((NP, _P), f32),
            pltpu.VMEM((2, _CHE, _P), f32),
            pltpu.VMEM((GPT, _G), jnp.int32),
            pltpu.VMEM((64, _P), f32),
            pltpu.SemaphoreType.DMA((2,)),
            pltpu.SemaphoreType.DMA((2,)),
        ],
    )(new_rows, idx2)

    flat = pl.pallas_call(
        _combine_body,
        out_shape=jax.ShapeDtypeStruct((NP, _P), f32),
    )(partial[:NP], partial[NP:])
    return flat[:num_nodes, :DH]


# speedup vs baseline: 2.5095x; 1.0061x over previous
import jax
import jax.numpy as jnp
from jax import lax
from jax.experimental import pallas as pl
from jax.experimental.pallas import tpu as pltpu
from jax.experimental.pallas import tpu_sc as plsc

_NC = 2
_NS = 16
_NW = _NC * _NS
_P = 16

_G = 128
_EPW = 10240
_CHE = 2560


def _mlp_body(x_ref, w1_ref, b1_ref, w2_ref, b2_ref, o_ref):
    h = jnp.dot(x_ref[...], w1_ref[...], preferred_element_type=jnp.float32)
    h = jnp.maximum(h + b1_ref[...], 0.0)
    o = jnp.dot(h, w2_ref[...], preferred_element_type=jnp.float32)
    o_ref[...] = o + b2_ref[...]


def _combine_body(a_ref, b_ref, o_ref):
    o_ref[...] = a_ref[...] + b_ref[...]


def kernel(nodes, edges, senders, receivers, bn_scale, bn_bias, bn_mean,
           bn_var, W1, b1, W2, b2):
    del senders
    f32 = jnp.float32
    num_nodes = nodes.shape[0]
    E, DE = edges.shape
    DH = W1.shape[1]

    s = bn_scale * lax.rsqrt(bn_var + 1e-5)
    t = bn_bias - bn_mean * s
    W1f = s[:, None] * W1
    b1f = b1 + t @ W1

    W1p = jnp.zeros((DE, _P), f32).at[:, :DH].set(W1f)
    W2p = jnp.zeros((_P, _P), f32).at[:DH, :DH].set(W2)
    b1p = jnp.zeros((1, _P), f32).at[0, :DH].set(b1f)
    b2p = jnp.zeros((1, _P), f32).at[0, :DH].set(b2)

    EP = _NW * _EPW
    assert E <= EP
    BLKE = 3200
    assert E % BLKE == 0
    new_rows = pl.pallas_call(
        _mlp_body,
        grid=(E // BLKE,),
        in_specs=[
            pl.BlockSpec((BLKE, DE), lambda i: (i, 0)),
            pl.BlockSpec((DE, _P), lambda i: (0, 0)),
            pl.BlockSpec((1, _P), lambda i: (0, 0)),
            pl.BlockSpec((_P, _P), lambda i: (0, 0)),
            pl.BlockSpec((1, _P), lambda i: (0, 0)),
        ],
        out_specs=pl.BlockSpec((BLKE, _P), lambda i: (i, 0)),
        out_shape=jax.ShapeDtypeStruct((EP, _P), f32),
    )(edges, W1p, b1p, W2p, b2p)

    NP = ((num_nodes + _NS * _P) // (_NS * _P)) * (_NS * _P)
    ZR = NP // _NS
    GPT = _EPW // _G
    NCHUNK = _EPW // _CHE
    GPC = _CHE // _G

    idx2 = jnp.concatenate(
        [receivers, jnp.full((EP - E,), num_nodes, jnp.int32)]
    ).reshape(EP // _G, _G)

    def sc_body(rows_hbm, idx_hbm, out_hbm, acc, rows_v, idx_v, zer_v,
                lsem, ssem):
        c = lax.axis_index("c")
        sub = lax.axis_index("s")
        wid = c * _NS + sub

        for i in range(64):
            zer_v[i, :] = jnp.zeros((_P,), f32)
        row0 = sub * ZR

        def zloop(j, carry):
            pltpu.sync_copy(zer_v, acc.at[pl.ds(row0 + j * 64, 64)])
            return carry

        lax.fori_loop(0, ZR // 64, zloop, 0)

        pltpu.sync_copy(idx_hbm.at[pl.ds(wid * GPT, GPT)], idx_v)
        plsc.subcore_barrier()

        ebase = wid * _EPW

        def load(ck, b):
            pltpu.make_async_copy(
                rows_hbm.at[pl.ds(ebase + ck * _CHE, _CHE)],
                rows_v.at[b], lsem.at[b]).start()

        def wait_load(b):
            pltpu.make_async_copy(
                rows_hbm.at[pl.ds(0, _CHE)], rows_v.at[b], lsem.at[b]).wait()

        def fire(ck, b):
            for j in range(GPC):
                pltpu.make_async_copy(
                    rows_v.at[b].at[pl.ds(j * _G, _G)],
                    acc.at[idx_v.at[ck * GPC + j]],
                    ssem.at[b]).start(add=True)

        def drain(ck, b):
            for j in range(GPC):
                pltpu.make_async_copy(
                    rows_v.at[b].at[pl.ds(j * _G, _G)],
                    acc.at[idx_v.at[ck * GPC + j]],
                    ssem.at[b]).wait()

        load(0, 0)
        for ck in range(NCHUNK):
            b = ck % 2
            wait_load(b)
            fire(ck, b)
            if ck + 1 < NCHUNK:
                if ck >= 1:
                    drain(ck - 1, 1 - b)
                load(ck + 1, 1 - b)
        if NCHUNK >= 2:
            drain(NCHUNK - 2, NCHUNK % 2)
        drain(NCHUNK - 1, (NCHUNK - 1) % 2)
        plsc.subcore_barrier()

        pltpu.sync_copy(acc.at[pl.ds(row0, ZR)],
                        out_hbm.at[pl.ds(c * NP + row0, ZR)])

    mesh = plsc.VectorSubcoreMesh(core_axis_name="c", subcore_axis_name="s")
    partial = pl.kernel(
        sc_body,
        out_type=jax.ShapeDtypeStruct((_NC * NP, _P), f32),
        mesh=mesh,
        compiler_params=pltpu.CompilerParams(use_tc_tiling_on_sc=False),
        scratch_types=[
            pltpu.VMEM_SHARED((NP, _P), f32),
            pltpu.VMEM((2, _CHE, _P), f32),
            pltpu.VMEM((GPT, _G), jnp.int32),
            pltpu.VMEM((64, _P), f32),
            pltpu.SemaphoreType.DMA((2,)),
            pltpu.SemaphoreType.DMA((2,)),
        ],
    )(new_rows, idx2)

    flat = pl.pallas_call(
        _combine_body,
        out_shape=jax.ShapeDtypeStruct((NP, _P), f32),
    )(partial[:NP], partial[NP:])
    return flat[:num_nodes, :DH]
